# baseline (speedup 1.0000x reference)
import functools

import jax
import jax.numpy as jnp
from jax import lax
from jax.experimental import pallas as pl
from jax.experimental.pallas import tpu as pltpu
from jax.experimental.pallas import tpu_sc as plsc

_N = 10000
_R = 5
_D = 128
_L = 16
_NC = 2
_NS = 16
_NW = _NC * _NS
_K = 80
_CNT_PAD = 81920
_SLICE = _CNT_PAD // _NS
_CAP = 10240
_TRASH = 10200
_NROWS = 10240
_EPT = 10000
_BN = 1000


def _lrelu(v):
    return jnp.where(v >= 0, v, 0.01 * v)


def _dotd(a, b):
    return jnp.dot(a, b, preferred_element_type=jnp.float32)


def _encoder(des, tweet, npc, W_des, W_tw, W_npc, b_pre, W_in, b_in):
    n = des.shape[0]
    nb = n // _BN

    def body(des_r, tw_r, npc_r, wd_r, wt_r, wn_r, bp_r, wi_r, bi_r, o_r):
        d = _dotd(des_r[...], wd_r[...])
        t = _dotd(tw_r[...], wt_r[...])
        nc = _dotd(npc_r[...], wn_r[...])
        xp = _lrelu(jnp.concatenate([d, t, nc], axis=1) + bp_r[...])
        o_r[...] = _lrelu(_dotd(xp, wi_r[...]) + bi_r[...])

    return pl.pallas_call(
        body,
        grid=(nb,),
        in_specs=[
            pl.BlockSpec((_BN, 768), lambda i: (i, 0)),
            pl.BlockSpec((_BN, 768), lambda i: (i, 0)),
            pl.BlockSpec((_BN, 128), lambda i: (i, 0)),
            pl.BlockSpec((768, 32), lambda i: (0, 0)),
            pl.BlockSpec((768, 32), lambda i: (0, 0)),
            pl.BlockSpec((128, 64), lambda i: (0, 0)),
            pl.BlockSpec((1, 128), lambda i: (0, 0)),
            pl.BlockSpec((128, 128), lambda i: (0, 0)),
            pl.BlockSpec((1, 128), lambda i: (0, 0)),
        ],
        out_specs=pl.BlockSpec((_BN, 128), lambda i: (i, 0)),
        out_shape=jax.ShapeDtypeStruct((n, 128), jnp.float32),
    )(des, tweet, npc, W_des, W_tw, W_npc, b_pre, W_in, b_in)


def _rgcn_combine(x, p0, p1, c0, c1, W_root, b, W_rel, head, W_o1, b_o1,
                  W_o2p, b_o2p):
    n = x.shape[0]
    nb = n // _BN

    def body(x_r, p0_r, p1_r, c0_r, c1_r, wr_r, b_r, wrel_r,
             w1_r, b1_r, w2_r, b2_r, o_r):
        cc = jnp.clip(c0_r[...] + c1_r[...], 1.0)
        acc = _dotd(x_r[...], wr_r[...]) + b_r[...]
        for r in range(_R):
            agg = (p0_r[r] + p1_r[r]) / cc[:, r:r + 1]
            acc = acc + _dotd(agg, wrel_r[r])
        if head:
            h = _lrelu(_dotd(acc, w1_r[...]) + b1_r[...])
            acc = _dotd(h, w2_r[...]) + b2_r[...]
        o_r[...] = acc

    return pl.pallas_call(
        body,
        grid=(nb,),
        in_specs=[
            pl.BlockSpec((_BN, 128), lambda i: (i, 0)),
            pl.BlockSpec((_R, _BN, 128), lambda i: (0, i, 0)),
            pl.BlockSpec((_R, _BN, 128), lambda i: (0, i, 0)),
            pl.BlockSpec((_BN, 8), lambda i: (i, 0)),
            pl.BlockSpec((_BN, 8), lambda i: (i, 0)),
            pl.BlockSpec((128, 128), lambda i: (0, 0)),
            pl.BlockSpec((1, 128), lambda i: (0, 0)),
            pl.BlockSpec((_R, 128, 128), lambda i: (0, 0, 0)),
            pl.BlockSpec((128, 128), lambda i: (0, 0)),
            pl.BlockSpec((1, 128), lambda i: (0, 0)),
            pl.BlockSpec((128, 128), lambda i: (0, 0)),
            pl.BlockSpec((1, 128), lambda i: (0, 0)),
        ],
        out_specs=pl.BlockSpec((_BN, 128), lambda i: (i, 0)),
        out_shape=jax.ShapeDtypeStruct((n, 128), jnp.float32),
    )(x, p0, p1, c0, c1, W_root, b, W_rel, W_o1, b_o1, W_o2p, b_o2p)


_MESH = dict(core_axis_name="c", subcore_axis_name="s")
_CPAR = pltpu.CompilerParams(needs_layout_passes=False)


def _sc_count(dst1, typ1):
    e_per = dst1.shape[0] // _NW

    @functools.partial(
        pl.kernel,
        out_type=jax.ShapeDtypeStruct((_NC * _CNT_PAD,), jnp.float32),
        mesh=plsc.VectorSubcoreMesh(**_MESH),
        scratch_types=[
            pltpu.VMEM((e_per,), jnp.int32),
            pltpu.VMEM((e_per,), jnp.int32),
            pltpu.VMEM((_K,), jnp.float32),
            pltpu.VMEM((_SLICE,), jnp.float32),
            pltpu.VMEM((1, _K), jnp.int32),
            pltpu.VMEM_SHARED((_CNT_PAD,), jnp.float32),
        ],
        compiler_params=_CPAR,
    )
    def run(dst_h, typ_h, out_h, dbuf, tbuf, ones, zbuf, ibuf, cnt_sh):
        c = lax.axis_index("c")
        s = lax.axis_index("s")

        @pl.loop(0, _SLICE // _L)
        def _(i):
            zbuf[pl.ds(i * _L, _L)] = jnp.zeros((_L,), jnp.float32)

        @pl.loop(0, _K // _L)
        def _(i):
            ones[pl.ds(i * _L, _L)] = jnp.ones((_L,), jnp.float32)

        pltpu.sync_copy(zbuf, cnt_sh.at[pl.ds(s * _SLICE, _SLICE)])
        plsc.subcore_barrier()

        tb = (c * _NS + s) * e_per
        pltpu.sync_copy(dst_h.at[pl.ds(tb, e_per)], dbuf)
        pltpu.sync_copy(typ_h.at[pl.ds(tb, e_per)], tbuf)

        @pl.loop(0, e_per // _L)
        def _(q):
            sl = pl.ds(q * _L, _L)
            tbuf[sl] = dbuf[sl] * 8 + tbuf[sl]

        @pl.loop(0, e_per // _K)
        def _(j):
            for g in range(_K // _L):
                ibuf[0, pl.ds(g * _L, _L)] = tbuf[pl.ds(j * _K + g * _L, _L)]
            pltpu.sync_copy(ones, cnt_sh.at[ibuf.at[0]], add=True)

        plsc.subcore_barrier()
        pltpu.sync_copy(cnt_sh.at[pl.ds(s * _SLICE, _SLICE)], zbuf)
        pltpu.sync_copy(zbuf, out_h.at[pl.ds(c * _CNT_PAD + s * _SLICE, _SLICE)])

    return run(dst1, typ1)


def _sc_bin(typ1):
    @functools.partial(
        pl.kernel,
        out_type=(jax.ShapeDtypeStruct((_NW * _R * _CAP,), jnp.int32),
                  jax.ShapeDtypeStruct((_NW * _L,), jnp.int32)),
        mesh=plsc.VectorSubcoreMesh(**_MESH),
        scratch_types=[
            pltpu.VMEM((_EPT,), jnp.int32),
            pltpu.VMEM((_R * _CAP,), jnp.int32),
            pltpu.VMEM((_R, _L), jnp.int32),
            pltpu.VMEM((_L,), jnp.int32),
        ],
        compiler_params=_CPAR,
    )
    def run(typ_h, perm_h, cnt_h, tbuf, perm_v, ctrv, cntv):
        c = lax.axis_index("c")
        s = lax.axis_index("s")
        wid = c * _NS + s
        pltpu.sync_copy(typ_h.at[pl.ds(wid * _EPT, _EPT)], tbuf)

        iota = lax.iota(jnp.int32, _L)
        zero = jnp.zeros((_L,), jnp.int32)
        for r in range(_R):
            ctrv[r, :] = zero

        @pl.loop(0, _EPT // _L)
        def _(q):
            t16 = tbuf[pl.ds(q * _L, _L)]
            eid = q * _L + iota
            for r in range(_R):
                m = t16 == r
                pos = plsc.cumsum(jnp.where(m, 1, 0))
                ctr16 = ctrv[r, :]
                slot = r * _CAP + ctr16 + pos - 1
                plsc.store_scatter(perm_v, [slot], eid, mask=m)
                ctrv[r, :] = ctr16 + pos[_L - 1]

        cv = zero
        for r in range(_R):
            ctr = ctrv[r, :][0]
            need = (_K - lax.rem(ctr, _K)) % _K
            for g in range(_K // _L):
                io = g * _L + iota
                slot = jnp.where(io < need, r * _CAP + ctr + io,
                                 r * _CAP + _CAP - _K + io)
                plsc.store_scatter(perm_v, [slot],
                                   jnp.full((_L,), _EPT, jnp.int32))
            cv = jnp.where(iota == r, ctr + need, cv)
        cntv[...] = cv

        pltpu.sync_copy(perm_v, perm_h.at[pl.ds(wid * _R * _CAP, _R * _CAP)])
        pltpu.sync_copy(cntv, cnt_h.at[pl.ds(wid * _L, _L)])

    return run(typ1)


def _sc_edge(x, perm, bsz, src1, dst1):
    @functools.partial(
        pl.kernel,
        out_type=jax.ShapeDtypeStruct((_NC, _R, _NROWS, _D), jnp.float32),
        mesh=plsc.VectorSubcoreMesh(**_MESH),
        scratch_types=[
            pltpu.VMEM((_EPT + _L,), jnp.int32),
            pltpu.VMEM((_EPT + _L,), jnp.int32),
            pltpu.VMEM((_NW * _L,), jnp.int32),
            pltpu.VMEM((_K,), jnp.int32),
            pltpu.VMEM((_K,), jnp.int32),
            pltpu.VMEM((1, _K), jnp.int32),
            pltpu.VMEM((_K, _D), jnp.float32),
            pltpu.VMEM((_K, _D), jnp.float32),
            pltpu.VMEM_SHARED((_NROWS, _D), jnp.float32),
            pltpu.SemaphoreType.DMA,
        ],
        compiler_params=_CPAR,
    )
    def run(x_h, perm_h, bsz_h, src_h, dst_h, out_h,
            sbuf, dbuf, cbuf, pbuf, gbuf, ibuf, rows, zrows, acc_sh, gsem):
        c = lax.axis_index("c")
        s = lax.axis_index("s")
        wid = c * _NS + s

        pltpu.sync_copy(src_h.at[pl.ds(wid * _EPT, _EPT)],
                        sbuf.at[pl.ds(0, _EPT)])
        pltpu.sync_copy(dst_h.at[pl.ds(wid * _EPT, _EPT)],
                        dbuf.at[pl.ds(0, _EPT)])
        sbuf[pl.ds(_EPT, _L)] = jnp.zeros((_L,), jnp.int32)
        dbuf[pl.ds(_EPT, _L)] = jnp.full((_L,), _TRASH, jnp.int32)
        pltpu.sync_copy(bsz_h, cbuf)
        szs = cbuf[pl.ds(wid * _L, _L)]

        @pl.loop(0, _K)
        def _(k):
            for i in range(_D // _L):
                zrows[k, pl.ds(i * _L, _L)] = jnp.zeros((_L,), jnp.float32)

        rpt = _NROWS // _NS
        for r in range(_R):
            for q in range(rpt // _K):
                pltpu.sync_copy(zrows, acc_sh.at[pl.ds(s * rpt + q * _K, _K)])
            plsc.subcore_barrier()

            nch = szs[r] // _K

            @pl.loop(0, nch)
            def _(j):
                base = (wid * _R + r) * _CAP + j * _K
                pltpu.sync_copy(perm_h.at[pl.ds(base, _K)], pbuf)
                for g in range(_K // _L):
                    sl = pl.ds(g * _L, _L)
                    p16 = pbuf[sl]
                    gbuf[sl] = plsc.load_gather(sbuf, [p16])
                    ibuf[0, sl] = plsc.load_gather(dbuf, [p16])
                pltpu.async_copy(x_h.at[gbuf], rows, gsem).wait()
                pltpu.sync_copy(rows, acc_sh.at[ibuf.at[0]], add=True)

            plsc.subcore_barrier()
            for q in range(rpt // _K):
                r0 = s * rpt + q * _K
                pltpu.sync_copy(acc_sh.at[pl.ds(r0, _K)], rows)
                pltpu.sync_copy(rows, out_h.at[c, r, pl.ds(r0, _K)])

    return run(x, perm, bsz, src1, dst1)


def kernel(des, tweet, num_prop, cat_prop, edge_index, edge_type,
           W_des, b_des, W_tw, b_tw, W_np, b_np, W_cp, b_cp,
           W_in, b_in, W_rel, W_root, b_rgcn, W_o1, b_o1, W_o2, b_o2):
    src1 = edge_index[0].astype(jnp.int32)
    dst1 = edge_index[1].astype(jnp.int32)
    typ1 = edge_type.astype(jnp.int32)

    npc = jnp.concatenate([num_prop, cat_prop], axis=1)
    npc = jnp.pad(npc, ((0, 0), (0, 128 - npc.shape[1])))
    nd = num_prop.shape[1]
    W_npc = jnp.zeros((128, 64), jnp.float32)
    W_npc = W_npc.at[:nd, :32].set(W_np).at[nd:nd + cat_prop.shape[1], 32:].set(W_cp)
    b_pre = jnp.concatenate([b_des, b_tw, b_np, b_cp]).reshape(1, 128)
    W_o2p = jnp.zeros((128, 128), jnp.float32).at[:, :W_o2.shape[1]].set(W_o2)
    b_o2p = jnp.zeros((128,), jnp.float32).at[:W_o2.shape[1]].set(b_o2).reshape(1, 128)
    b_rg = b_rgcn.reshape(1, 128)

    cnts = _sc_count(dst1, typ1)
    c0 = cnts[:_CNT_PAD].reshape(_CNT_PAD // 8, 8)[:_N]
    c1 = cnts[_CNT_PAD:].reshape(_CNT_PAD // 8, 8)[:_N]
    perm, bsz = _sc_bin(typ1)

    x1 = _encoder(des, tweet, npc, W_des, W_tw, W_npc, b_pre,
                  W_in, b_in.reshape(1, 128))
    p1 = _sc_edge(x1, perm, bsz, src1, dst1)
    x2 = _rgcn_combine(x1, p1[0], p1[1], c0, c1, W_root, b_rg, W_rel,
                       False, W_o1, b_o1.reshape(1, 128), W_o2p, b_o2p)
    p2 = _sc_edge(x2, perm, bsz, src1, dst1)
    out = _rgcn_combine(x2, p2[0], p2[1], c0, c1, W_root, b_rg, W_rel,
                        True, W_o1, b_o1.reshape(1, 128), W_o2p, b_o2p)
    return out[:, :W_o2.shape[1]]


# speedup vs baseline: 1.1138x; 1.1138x over previous
import functools

import jax
import jax.numpy as jnp
from jax import lax
from jax.experimental import pallas as pl
from jax.experimental.pallas import tpu as pltpu
from jax.experimental.pallas import tpu_sc as plsc

_N = 10000
_R = 5
_D = 128
_L = 16
_NC = 2
_NS = 16
_NW = _NC * _NS
_K = 80
_CNT_PAD = 81920
_SLICE = _CNT_PAD // _NS
_CAP = 10240
_TRASH = 10200
_NROWS = 10240
_EPT = 10000
_BN = 1000


def _lrelu(v):
    return jnp.where(v >= 0, v, 0.01 * v)


def _dotd(a, b):
    return jnp.dot(a, b, preferred_element_type=jnp.float32)


def _encoder(des, tweet, npc, W_des, W_tw, W_npc, b_pre, W_in, b_in):
    n = des.shape[0]
    nb = n // _BN

    def body(des_r, tw_r, npc_r, wd_r, wt_r, wn_r, bp_r, wi_r, bi_r, o_r):
        d = _dotd(des_r[...], wd_r[...])
        t = _dotd(tw_r[...], wt_r[...])
        nc = _dotd(npc_r[...], wn_r[...])
        xp = _lrelu(jnp.concatenate([d, t, nc], axis=1) + bp_r[...])
        o_r[...] = _lrelu(_dotd(xp, wi_r[...]) + bi_r[...])

    return pl.pallas_call(
        body,
        grid=(nb,),
        in_specs=[
            pl.BlockSpec((_BN, 768), lambda i: (i, 0)),
            pl.BlockSpec((_BN, 768), lambda i: (i, 0)),
            pl.BlockSpec((_BN, 128), lambda i: (i, 0)),
            pl.BlockSpec((768, 32), lambda i: (0, 0)),
            pl.BlockSpec((768, 32), lambda i: (0, 0)),
            pl.BlockSpec((128, 64), lambda i: (0, 0)),
            pl.BlockSpec((1, 128), lambda i: (0, 0)),
            pl.BlockSpec((128, 128), lambda i: (0, 0)),
            pl.BlockSpec((1, 128), lambda i: (0, 0)),
        ],
        out_specs=pl.BlockSpec((_BN, 128), lambda i: (i, 0)),
        out_shape=jax.ShapeDtypeStruct((n, 128), jnp.float32),
    )(des, tweet, npc, W_des, W_tw, W_npc, b_pre, W_in, b_in)


def _rgcn_combine(x, p0, p1, c0, c1, W_root, b, W_rel, head, W_o1, b_o1,
                  W_o2p, b_o2p):
    n = x.shape[0]
    nb = n // _BN

    def body(x_r, p0_r, p1_r, c0_r, c1_r, wr_r, b_r, wrel_r,
             w1_r, b1_r, w2_r, b2_r, o_r):
        cc = jnp.clip(c0_r[...] + c1_r[...], 1.0)
        acc = _dotd(x_r[...], wr_r[...]) + b_r[...]
        for r in range(_R):
            agg = (p0_r[r] + p1_r[r]) / cc[:, r:r + 1]
            acc = acc + _dotd(agg, wrel_r[r])
        if head:
            h = _lrelu(_dotd(acc, w1_r[...]) + b1_r[...])
            acc = _dotd(h, w2_r[...]) + b2_r[...]
        o_r[...] = acc

    return pl.pallas_call(
        body,
        grid=(nb,),
        in_specs=[
            pl.BlockSpec((_BN, 128), lambda i: (i, 0)),
            pl.BlockSpec((_R, _BN, 128), lambda i: (0, i, 0)),
            pl.BlockSpec((_R, _BN, 128), lambda i: (0, i, 0)),
            pl.BlockSpec((_BN, 8), lambda i: (i, 0)),
            pl.BlockSpec((_BN, 8), lambda i: (i, 0)),
            pl.BlockSpec((128, 128), lambda i: (0, 0)),
            pl.BlockSpec((1, 128), lambda i: (0, 0)),
            pl.BlockSpec((_R, 128, 128), lambda i: (0, 0, 0)),
            pl.BlockSpec((128, 128), lambda i: (0, 0)),
            pl.BlockSpec((1, 128), lambda i: (0, 0)),
            pl.BlockSpec((128, 128), lambda i: (0, 0)),
            pl.BlockSpec((1, 128), lambda i: (0, 0)),
        ],
        out_specs=pl.BlockSpec((_BN, 128), lambda i: (i, 0)),
        out_shape=jax.ShapeDtypeStruct((n, 128), jnp.float32),
    )(x, p0, p1, c0, c1, W_root, b, W_rel, W_o1, b_o1, W_o2p, b_o2p)


_MESH = dict(core_axis_name="c", subcore_axis_name="s")
_CPAR = pltpu.CompilerParams(needs_layout_passes=False)


def _sc_count(dst1, typ1):
    e_per = dst1.shape[0] // _NW

    @functools.partial(
        pl.kernel,
        out_type=jax.ShapeDtypeStruct((_NC * _CNT_PAD,), jnp.float32),
        mesh=plsc.VectorSubcoreMesh(**_MESH),
        scratch_types=[
            pltpu.VMEM((e_per,), jnp.int32),
            pltpu.VMEM((e_per,), jnp.int32),
            pltpu.VMEM((_K,), jnp.float32),
            pltpu.VMEM((_SLICE,), jnp.float32),
            pltpu.VMEM((1, _K), jnp.int32),
            pltpu.VMEM_SHARED((_CNT_PAD,), jnp.float32),
        ],
        compiler_params=_CPAR,
    )
    def run(dst_h, typ_h, out_h, dbuf, tbuf, ones, zbuf, ibuf, cnt_sh):
        c = lax.axis_index("c")
        s = lax.axis_index("s")

        @pl.loop(0, _SLICE // _L)
        def _(i):
            zbuf[pl.ds(i * _L, _L)] = jnp.zeros((_L,), jnp.float32)

        @pl.loop(0, _K // _L)
        def _(i):
            ones[pl.ds(i * _L, _L)] = jnp.ones((_L,), jnp.float32)

        pltpu.sync_copy(zbuf, cnt_sh.at[pl.ds(s * _SLICE, _SLICE)])
        plsc.subcore_barrier()

        tb = (c * _NS + s) * e_per
        pltpu.sync_copy(dst_h.at[pl.ds(tb, e_per)], dbuf)
        pltpu.sync_copy(typ_h.at[pl.ds(tb, e_per)], tbuf)

        @pl.loop(0, e_per // _L)
        def _(q):
            sl = pl.ds(q * _L, _L)
            tbuf[sl] = dbuf[sl] * 8 + tbuf[sl]

        @pl.loop(0, e_per // _K)
        def _(j):
            for g in range(_K // _L):
                ibuf[0, pl.ds(g * _L, _L)] = tbuf[pl.ds(j * _K + g * _L, _L)]
            pltpu.sync_copy(ones, cnt_sh.at[ibuf.at[0]], add=True)

        plsc.subcore_barrier()
        pltpu.sync_copy(cnt_sh.at[pl.ds(s * _SLICE, _SLICE)], zbuf)
        pltpu.sync_copy(zbuf, out_h.at[pl.ds(c * _CNT_PAD + s * _SLICE, _SLICE)])

    return run(dst1, typ1)


def _sc_bin(typ1):
    @functools.partial(
        pl.kernel,
        out_type=(jax.ShapeDtypeStruct((_NW * _R * _CAP,), jnp.int32),
                  jax.ShapeDtypeStruct((_NW * _L,), jnp.int32)),
        mesh=plsc.VectorSubcoreMesh(**_MESH),
        scratch_types=[
            pltpu.VMEM((_EPT,), jnp.int32),
            pltpu.VMEM((_R * _CAP,), jnp.int32),
            pltpu.VMEM((_R, _L), jnp.int32),
            pltpu.VMEM((_L,), jnp.int32),
        ],
        compiler_params=_CPAR,
    )
    def run(typ_h, perm_h, cnt_h, tbuf, perm_v, ctrv, cntv):
        c = lax.axis_index("c")
        s = lax.axis_index("s")
        wid = c * _NS + s
        pltpu.sync_copy(typ_h.at[pl.ds(wid * _EPT, _EPT)], tbuf)

        iota = lax.iota(jnp.int32, _L)
        zero = jnp.zeros((_L,), jnp.int32)
        for r in range(_R):
            ctrv[r, :] = zero

        @pl.loop(0, _EPT // _L)
        def _(q):
            t16 = tbuf[pl.ds(q * _L, _L)]
            eid = q * _L + iota
            for r in range(_R):
                m = t16 == r
                pos = plsc.cumsum(jnp.where(m, 1, 0))
                ctr16 = ctrv[r, :]
                slot = r * _CAP + ctr16 + pos - 1
                plsc.store_scatter(perm_v, [slot], eid, mask=m)
                ctrv[r, :] = ctr16 + pos[_L - 1]

        cv = zero
        for r in range(_R):
            ctr = ctrv[r, :][0]
            need = (_K - lax.rem(ctr, _K)) % _K
            for g in range(_K // _L):
                io = g * _L + iota
                slot = jnp.where(io < need, r * _CAP + ctr + io,
                                 r * _CAP + _CAP - _K + io)
                plsc.store_scatter(perm_v, [slot],
                                   jnp.full((_L,), _EPT, jnp.int32))
            cv = jnp.where(iota == r, ctr + need, cv)
        cntv[...] = cv

        pltpu.sync_copy(perm_v, perm_h.at[pl.ds(wid * _R * _CAP, _R * _CAP)])
        pltpu.sync_copy(cntv, cnt_h.at[pl.ds(wid * _L, _L)])

    return run(typ1)


def _sc_edge(x, perm, bsz, src1, dst1):
    @functools.partial(
        pl.kernel,
        out_type=jax.ShapeDtypeStruct((_NC, _R, _NROWS, _D), jnp.float32),
        mesh=plsc.VectorSubcoreMesh(**_MESH),
        scratch_types=[
            pltpu.VMEM((_EPT + _L,), jnp.int32),
            pltpu.VMEM((_EPT + _L,), jnp.int32),
            pltpu.VMEM((_NW * _L,), jnp.int32),
            pltpu.VMEM((2 * _K,), jnp.int32),
            pltpu.VMEM((_K,), jnp.int32),
            pltpu.VMEM((_K,), jnp.int32),
            pltpu.VMEM((2, _K), jnp.int32),
            pltpu.VMEM((_K, _D), jnp.float32),
            pltpu.VMEM((_K, _D), jnp.float32),
            pltpu.VMEM_SHARED((_NROWS, _D), jnp.float32),
            pltpu.SemaphoreType.DMA,
            pltpu.SemaphoreType.DMA,
            pltpu.SemaphoreType.DMA,
            pltpu.SemaphoreType.DMA,
        ],
        compiler_params=_CPAR,
    )
    def run(x_h, perm_h, bsz_h, src_h, dst_h, out_h,
            sbuf, dbuf, cbuf, pbuf, gbA, gbB, ibuf, rowsA, rowsB, acc_sh,
            gsA, gsB, ssA, ssB):
        c = lax.axis_index("c")
        s = lax.axis_index("s")
        wid = c * _NS + s

        pltpu.sync_copy(src_h.at[pl.ds(wid * _EPT, _EPT)],
                        sbuf.at[pl.ds(0, _EPT)])
        pltpu.sync_copy(dst_h.at[pl.ds(wid * _EPT, _EPT)],
                        dbuf.at[pl.ds(0, _EPT)])
        sbuf[pl.ds(_EPT, _L)] = jnp.zeros((_L,), jnp.int32)
        dbuf[pl.ds(_EPT, _L)] = jnp.full((_L,), _TRASH, jnp.int32)
        pltpu.sync_copy(bsz_h, cbuf)
        szs = cbuf[pl.ds(wid * _L, _L)]

        def _zero_rowsA():
            @pl.loop(0, _K)
            def _(k):
                for i in range(_D // _L):
                    rowsA[k, pl.ds(i * _L, _L)] = jnp.zeros((_L,), jnp.float32)

        def _indices(half, gb, irow):
            for g in range(_K // _L):
                sl = pl.ds(g * _L, _L)
                p16 = pbuf[pl.ds(half * _K + g * _L, _L)]
                gb[sl] = plsc.load_gather(sbuf, [p16])
                ibuf[irow, sl] = plsc.load_gather(dbuf, [p16])

        rpt = _NROWS // _NS
        for r in range(_R):
            _zero_rowsA()
            for q in range(rpt // _K):
                pltpu.sync_copy(rowsA, acc_sh.at[pl.ds(s * rpt + q * _K, _K)])
            plsc.subcore_barrier()

            nch = szs[r] // _K
            base = (wid * _R + r) * _CAP

            @pl.loop(0, nch // 2)
            def _(p):
                pltpu.sync_copy(perm_h.at[pl.ds(base + 2 * p * _K, 2 * _K)],
                                pbuf)
                _indices(0, gbA, 0)
                _indices(1, gbB, 1)
                dgA = pltpu.async_copy(x_h.at[gbA], rowsA, gsA)
                dgB = pltpu.async_copy(x_h.at[gbB], rowsB, gsB)
                dgA.wait()
                dsA = pltpu.async_copy(rowsA, acc_sh.at[ibuf.at[0]], ssA,
                                       add=True)
                dgB.wait()
                dsB = pltpu.async_copy(rowsB, acc_sh.at[ibuf.at[1]], ssB,
                                       add=True)
                dsA.wait()
                dsB.wait()

            @pl.when(lax.rem(nch, 2) == 1)
            def _():
                pltpu.sync_copy(perm_h.at[pl.ds(base + (nch - 1) * _K, _K)],
                                pbuf.at[pl.ds(0, _K)])
                _indices(0, gbA, 0)
                pltpu.async_copy(x_h.at[gbA], rowsA, gsA).wait()
                pltpu.sync_copy(rowsA, acc_sh.at[ibuf.at[0]], add=True)

            plsc.subcore_barrier()
            for q in range(rpt // _K):
                r0 = s * rpt + q * _K
                pltpu.sync_copy(acc_sh.at[pl.ds(r0, _K)], rowsA)
                pltpu.sync_copy(rowsA, out_h.at[c, r, pl.ds(r0, _K)])

    return run(x, perm, bsz, src1, dst1)


def kernel(des, tweet, num_prop, cat_prop, edge_index, edge_type,
           W_des, b_des, W_tw, b_tw, W_np, b_np, W_cp, b_cp,
           W_in, b_in, W_rel, W_root, b_rgcn, W_o1, b_o1, W_o2, b_o2):
    src1 = edge_index[0].astype(jnp.int32)
    dst1 = edge_index[1].astype(jnp.int32)
    typ1 = edge_type.astype(jnp.int32)

    npc = jnp.concatenate([num_prop, cat_prop], axis=1)
    npc = jnp.pad(npc, ((0, 0), (0, 128 - npc.shape[1])))
    nd = num_prop.shape[1]
    W_npc = jnp.zeros((128, 64), jnp.float32)
    W_npc = W_npc.at[:nd, :32].set(W_np).at[nd:nd + cat_prop.shape[1], 32:].set(W_cp)
    b_pre = jnp.concatenate([b_des, b_tw, b_np, b_cp]).reshape(1, 128)
    W_o2p = jnp.zeros((128, 128), jnp.float32).at[:, :W_o2.shape[1]].set(W_o2)
    b_o2p = jnp.zeros((128,), jnp.float32).at[:W_o2.shape[1]].set(b_o2).reshape(1, 128)
    b_rg = b_rgcn.reshape(1, 128)

    cnts = _sc_count(dst1, typ1)
    c0 = cnts[:_CNT_PAD].reshape(_CNT_PAD // 8, 8)[:_N]
    c1 = cnts[_CNT_PAD:].reshape(_CNT_PAD // 8, 8)[:_N]
    perm, bsz = _sc_bin(typ1)

    x1 = _encoder(des, tweet, npc, W_des, W_tw, W_npc, b_pre,
                  W_in, b_in.reshape(1, 128))
    p1 = _sc_edge(x1, perm, bsz, src1, dst1)
    x2 = _rgcn_combine(x1, p1[0], p1[1], c0, c1, W_root, b_rg, W_rel,
                       False, W_o1, b_o1.reshape(1, 128), W_o2p, b_o2p)
    p2 = _sc_edge(x2, perm, bsz, src1, dst1)
    out = _rgcn_combine(x2, p2[0], p2[1], c0, c1, W_root, b_rg, W_rel,
                        True, W_o1, b_o1.reshape(1, 128), W_o2p, b_o2p)
    return out[:, :W_o2.shape[1]]
